# baseline (speedup 1.0000x reference)
import functools

import jax
import jax.numpy as jnp
from jax import lax
from jax.experimental import pallas as pl
from jax.experimental.pallas import tpu as pltpu
from jax.experimental.pallas import tpu_sc as plsc

_L = 128
_NC = 2
_NS = 16
_NW = _NC * _NS
_HPC = 2


@functools.lru_cache(maxsize=None)
def _make_gather(b: int, h: int, d: int):
    assert b == _NW * _L and d % 8 == 0
    e8 = d // 8
    bt = b // _L
    n_chunks = h // _HPC
    n_loop = (n_chunks + 1) // 2

    mesh = plsc.VectorSubcoreMesh(core_axis_name="c", subcore_axis_name="s")

    @functools.partial(
        pl.kernel,
        mesh=mesh,
        out_type=jax.ShapeDtypeStruct((h, e8, bt, 8, _L), jnp.float32),
        scratch_types=[
            pltpu.VMEM((h * 4, _L // 4), jnp.int32),
            pltpu.VMEM((_HPC * _L, d), jnp.float32),
            pltpu.VMEM((_HPC * _L, d), jnp.float32),
            pltpu.VMEM((_HPC, e8, 8, _L), jnp.float32),
            pltpu.VMEM((_HPC, e8, 8, _L), jnp.float32),
            pltpu.SemaphoreType.DMA,
            pltpu.SemaphoreType.DMA,
            pltpu.SemaphoreType.DMA,
            pltpu.SemaphoreType.DMA,
        ],
        compiler_params=pltpu.CompilerParams(
            use_tc_tiling_on_sc=False, needs_layout_passes=False
        ),
    )
    def gather_kernel(table_hbm, idx_hbm, out_hbm, idx_v,
                      ib0, ib1, ob0, ob1, g0, g1, o0, o1):
        wid = lax.axis_index("s") * _NC + lax.axis_index("c")
        lane = lax.iota(jnp.int32, 16)
        ibs = (ib0, ib1)
        obs = (ob0, ob1)
        gsems = (g0, g1)
        osems = (o0, o1)

        pltpu.sync_copy(idx_hbm.at[wid], idx_v)

        q = _L // 4

        def fire_gathers(ci, p):
            for hh in range(_HPC):
                for k in range(4):
                    pltpu.async_copy(
                        table_hbm.at[idx_v.at[(ci * _HPC + hh) * 4 + k]],
                        ibs[p].at[pl.ds(hh * _L + k * q, q)],
                        gsems[p],
                    )

        def wait_gathers(ci, p):
            for hh in range(_HPC):
                for k in range(4):
                    pltpu.make_async_copy(
                        table_hbm.at[idx_v.at[(ci * _HPC + hh) * 4 + k]],
                        ibs[p].at[pl.ds(hh * _L + k * q, q)],
                        gsems[p],
                    ).wait()

        def fire_writes(ci, p):
            for hh in range(_HPC):
                pltpu.async_copy(
                    obs[p].at[hh],
                    out_hbm.at[ci * _HPC + hh, :, wid],
                    osems[p],
                )

        def wait_writes(ci, p):
            for hh in range(_HPC):
                pltpu.make_async_copy(
                    obs[p].at[hh],
                    out_hbm.at[ci * _HPC + hh, :, wid],
                    osems[p],
                ).wait()

        def retile(p):
            ib, ob = ibs[p], obs[p]

            def diag(d0, carry):
                cols = jnp.bitwise_and(lane + d0, d - 1)
                et_v = jnp.right_shift(cols, 3)
                es_v = jnp.bitwise_and(cols, 7)
                for hh in range(_HPC):
                    hh_v = jnp.full((16,), hh, jnp.int32)
                    for blg in range(_L // 16):
                        rows = hh * _L + blg * 16 + lane
                        vals = plsc.load_gather(ib, [rows, cols])
                        plsc.store_scatter(
                            ob, [hh_v, et_v, es_v, blg * 16 + lane], vals
                        )
                return carry

            lax.fori_loop(0, d, diag, 0, unroll=4)

        fire_gathers(0, 0)
        fire_gathers(1, 1)

        def body(j, carry):
            c0 = 2 * j
            c1 = c0 + 1
            wait_gathers(c0, 0)

            @pl.when(j > 0)
            def _():
                wait_writes(c0 - 2, 0)

            retile(0)
            fire_writes(c0, 0)

            @pl.when(c0 + 2 < n_chunks)
            def _():
                fire_gathers(c0 + 2, 0)

            @pl.when(c1 < n_chunks)
            def _():
                wait_gathers(c1, 1)

                @pl.when(j > 0)
                def _():
                    wait_writes(c1 - 2, 1)

                retile(1)
                fire_writes(c1, 1)

                @pl.when(c1 + 2 < n_chunks)
                def _():
                    fire_gathers(c1 + 2, 1)

            return carry

        lax.fori_loop(0, n_loop, body, 0)

        wait_writes(n_chunks - 2, 1)
        wait_writes(n_chunks - 1, 0)

    return gather_kernel


def kernel(table, input_ids):
    b, h = input_ids.shape
    d = table.shape[1]
    idx3 = (
        input_ids.T.reshape(h, _NW, _L)
        .transpose(1, 0, 2)
        .reshape(_NW, h * 4, _L // 4)
        .astype(jnp.int32)
    )
    out5 = _make_gather(b, h, d)(table, idx3)
    return out5.transpose(2, 4, 0, 1, 3).reshape(b, h, d)


# speedup vs baseline: 1.0065x; 1.0065x over previous
import functools

import jax
import jax.numpy as jnp
from jax import lax
from jax.experimental import pallas as pl
from jax.experimental.pallas import tpu as pltpu
from jax.experimental.pallas import tpu_sc as plsc

_L = 128
_NC = 2
_NS = 16
_NW = _NC * _NS
_HPC = 2


@functools.lru_cache(maxsize=None)
def _make_gather(b: int, h: int, d: int):
    assert b == _NW * _L and d % 8 == 0
    e8 = d // 8
    bt = b // _L
    n_chunks = h // _HPC
    n_loop = (n_chunks + 1) // 2

    mesh = plsc.VectorSubcoreMesh(core_axis_name="c", subcore_axis_name="s")

    @functools.partial(
        pl.kernel,
        mesh=mesh,
        out_type=jax.ShapeDtypeStruct((h, e8, bt, 8, _L), jnp.float32),
        scratch_types=[
            pltpu.VMEM((h * 2, _L // 2), jnp.int32),
            pltpu.VMEM((_HPC * _L, d), jnp.float32),
            pltpu.VMEM((_HPC * _L, d), jnp.float32),
            pltpu.VMEM((_HPC, e8, 8, _L), jnp.float32),
            pltpu.VMEM((_HPC, e8, 8, _L), jnp.float32),
            pltpu.SemaphoreType.DMA,
            pltpu.SemaphoreType.DMA,
            pltpu.SemaphoreType.DMA,
            pltpu.SemaphoreType.DMA,
        ],
        compiler_params=pltpu.CompilerParams(
            use_tc_tiling_on_sc=False, needs_layout_passes=False
        ),
    )
    def gather_kernel(table_hbm, idx_hbm, out_hbm, idx_v,
                      ib0, ib1, ob0, ob1, g0, g1, o0, o1):
        wid = lax.axis_index("s") * _NC + lax.axis_index("c")
        lane = lax.iota(jnp.int32, 16)
        ibs = (ib0, ib1)
        obs = (ob0, ob1)
        gsems = (g0, g1)
        osems = (o0, o1)

        pltpu.sync_copy(idx_hbm.at[wid], idx_v)

        _NSPLIT = 2
        q = _L // _NSPLIT

        def fire_gathers(ci, p):
            for hh in range(_HPC):
                for k in range(_NSPLIT):
                    pltpu.async_copy(
                        table_hbm.at[idx_v.at[(ci * _HPC + hh) * _NSPLIT + k]],
                        ibs[p].at[pl.ds(hh * _L + k * q, q)],
                        gsems[p],
                    )

        def wait_gathers(ci, p):
            for hh in range(_HPC):
                for k in range(_NSPLIT):
                    pltpu.make_async_copy(
                        table_hbm.at[idx_v.at[(ci * _HPC + hh) * _NSPLIT + k]],
                        ibs[p].at[pl.ds(hh * _L + k * q, q)],
                        gsems[p],
                    ).wait()

        def fire_writes(ci, p):
            for hh in range(_HPC):
                pltpu.async_copy(
                    obs[p].at[hh],
                    out_hbm.at[ci * _HPC + hh, :, wid],
                    osems[p],
                )

        def wait_writes(ci, p):
            for hh in range(_HPC):
                pltpu.make_async_copy(
                    obs[p].at[hh],
                    out_hbm.at[ci * _HPC + hh, :, wid],
                    osems[p],
                ).wait()

        def retile(p):
            ib, ob = ibs[p], obs[p]

            def diag(d0, carry):
                cols = jnp.bitwise_and(lane + d0, d - 1)
                et_v = jnp.right_shift(cols, 3)
                es_v = jnp.bitwise_and(cols, 7)
                for hh in range(_HPC):
                    hh_v = jnp.full((16,), hh, jnp.int32)
                    for blg in range(_L // 16):
                        rows = hh * _L + blg * 16 + lane
                        vals = plsc.load_gather(ib, [rows, cols])
                        plsc.store_scatter(
                            ob, [hh_v, et_v, es_v, blg * 16 + lane], vals
                        )
                return carry

            lax.fori_loop(0, d, diag, 0, unroll=4)

        fire_gathers(0, 0)
        fire_gathers(1, 1)

        def body(j, carry):
            c0 = 2 * j
            c1 = c0 + 1
            wait_gathers(c0, 0)

            @pl.when(j > 0)
            def _():
                wait_writes(c0 - 2, 0)

            retile(0)
            fire_writes(c0, 0)

            @pl.when(c0 + 2 < n_chunks)
            def _():
                fire_gathers(c0 + 2, 0)

            @pl.when(c1 < n_chunks)
            def _():
                wait_gathers(c1, 1)

                @pl.when(j > 0)
                def _():
                    wait_writes(c1 - 2, 1)

                retile(1)
                fire_writes(c1, 1)

                @pl.when(c1 + 2 < n_chunks)
                def _():
                    fire_gathers(c1 + 2, 1)

            return carry

        lax.fori_loop(0, n_loop, body, 0)

        wait_writes(n_chunks - 2, 1)
        wait_writes(n_chunks - 1, 0)

    return gather_kernel


def kernel(table, input_ids):
    b, h = input_ids.shape
    d = table.shape[1]
    idx3 = (
        input_ids.T.reshape(h, _NW, _L)
        .transpose(1, 0, 2)
        .reshape(_NW, h * 2, _L // 2)
        .astype(jnp.int32)
    )
    out5 = _make_gather(b, h, d)(table, idx3)
    return out5.transpose(2, 4, 0, 1, 3).reshape(b, h, d)
